# Initial kernel scaffold; baseline (speedup 1.0000x reference)
import jax
import jax.numpy as jnp
from jax.experimental import pallas as pl


def kernel(x, residual, norm_weight, w_fp8, input_scale, w_scale):
    raise NotImplementedError("write your pallas kernel here")


# speedup vs baseline: 1.6077x; 1.6077x over previous
import jax
import jax.numpy as jnp
from jax import lax
from jax.experimental import pallas as pl
from jax.experimental.pallas import tpu as pltpu

_EPS = 1e-5
_FP8_MAX = 448.0
_F8 = jnp.float8_e4m3fn


def _norm_quant_kernel(x_ref, r_ref, nw_ref, add_ref, q_ref):
    a = x_ref[...] + r_ref[...]
    add_ref[...] = a
    inv = lax.rsqrt(jnp.mean(a * a, axis=-1, keepdims=True) + _EPS)
    v = a * inv * nw_ref[...]
    q_ref[...] = jnp.clip(v, -_FP8_MAX, _FP8_MAX).astype(_F8)


def _matmul_kernel(q_ref, w_ref, s_ref, o_ref):
    acc = lax.dot_general(
        q_ref[...],
        w_ref[...],
        dimension_numbers=(((1,), (1,)), ((), ())),
        preferred_element_type=jnp.float32,
    )
    o_ref[...] = acc * s_ref[0]


def kernel(x, residual, norm_weight, w_fp8, input_scale, w_scale):
    n, d_in = x.shape
    d_out = w_fp8.shape[0]

    w8 = w_fp8.astype(_F8)
    nw = (norm_weight / input_scale).astype(jnp.float32).reshape(1, d_in)
    out_scale = (input_scale * w_scale).astype(jnp.float32).reshape(1)

    bm1 = min(256, n)
    add_out, q8 = pl.pallas_call(
        _norm_quant_kernel,
        grid=(n // bm1,),
        in_specs=[
            pl.BlockSpec((bm1, d_in), lambda i: (i, 0)),
            pl.BlockSpec((bm1, d_in), lambda i: (i, 0)),
            pl.BlockSpec((1, d_in), lambda i: (0, 0)),
        ],
        out_specs=[
            pl.BlockSpec((bm1, d_in), lambda i: (i, 0)),
            pl.BlockSpec((bm1, d_in), lambda i: (i, 0)),
        ],
        out_shape=[
            jax.ShapeDtypeStruct((n, d_in), jnp.float32),
            jax.ShapeDtypeStruct((n, d_in), _F8),
        ],
        compiler_params=pltpu.CompilerParams(
            dimension_semantics=("parallel",),
            vmem_limit_bytes=56 * 1024 * 1024,
        ),
    )(x, residual, nw)

    bm2 = min(2048, n)
    bn2 = min(1024, d_out)
    out = pl.pallas_call(
        _matmul_kernel,
        grid=(n // bm2, d_out // bn2),
        in_specs=[
            pl.BlockSpec((bm2, d_in), lambda i, j: (i, 0)),
            pl.BlockSpec((bn2, d_in), lambda i, j: (j, 0)),
            pl.BlockSpec(memory_space=pltpu.SMEM),
        ],
        out_specs=pl.BlockSpec((bm2, bn2), lambda i, j: (i, j)),
        out_shape=jax.ShapeDtypeStruct((n, d_out), jnp.float32),
        compiler_params=pltpu.CompilerParams(
            dimension_semantics=("parallel", "arbitrary"),
            vmem_limit_bytes=56 * 1024 * 1024,
        ),
    )(q8, w8, out_scale)

    return (out, add_out)


# speedup vs baseline: 1.6793x; 1.0445x over previous
import jax
import jax.numpy as jnp
from jax import lax
from jax.experimental import pallas as pl
from jax.experimental.pallas import tpu as pltpu

_EPS = 1e-5
_FP8_MAX = 448.0
_F8 = jnp.float8_e4m3fn


def _norm_quant_kernel(x_ref, r_ref, nw_ref, w_ref, add_ref, q_ref, w8_ref):
    a = x_ref[...] + r_ref[...]
    add_ref[...] = a
    inv = lax.rsqrt(jnp.mean(a * a, axis=-1, keepdims=True) + _EPS)
    v = a * inv * nw_ref[...]
    q_ref[...] = jnp.clip(v, -_FP8_MAX, _FP8_MAX).astype(_F8)
    w8_ref[...] = w_ref[...].astype(_F8)


def _matmul_kernel(q_ref, w_ref, s_ref, o_ref):
    acc = lax.dot_general(
        q_ref[...],
        w_ref[...],
        dimension_numbers=(((1,), (1,)), ((), ())),
        preferred_element_type=jnp.float32,
    )
    o_ref[...] = acc * s_ref[0]


def kernel(x, residual, norm_weight, w_fp8, input_scale, w_scale):
    n, d_in = x.shape
    d_out = w_fp8.shape[0]

    nw = (norm_weight / input_scale).astype(jnp.float32).reshape(1, d_in)
    out_scale = (input_scale * w_scale).astype(jnp.float32).reshape(1)

    bm1 = min(256, n)
    g1 = n // bm1
    bw1 = d_out // g1
    add_out, q8, w8 = pl.pallas_call(
        _norm_quant_kernel,
        grid=(g1,),
        in_specs=[
            pl.BlockSpec((bm1, d_in), lambda i: (i, 0)),
            pl.BlockSpec((bm1, d_in), lambda i: (i, 0)),
            pl.BlockSpec((1, d_in), lambda i: (0, 0)),
            pl.BlockSpec((bw1, d_in), lambda i: (i, 0)),
        ],
        out_specs=[
            pl.BlockSpec((bm1, d_in), lambda i: (i, 0)),
            pl.BlockSpec((bm1, d_in), lambda i: (i, 0)),
            pl.BlockSpec((bw1, d_in), lambda i: (i, 0)),
        ],
        out_shape=[
            jax.ShapeDtypeStruct((n, d_in), jnp.float32),
            jax.ShapeDtypeStruct((n, d_in), _F8),
            jax.ShapeDtypeStruct((d_out, d_in), _F8),
        ],
        compiler_params=pltpu.CompilerParams(
            dimension_semantics=("parallel",),
            vmem_limit_bytes=56 * 1024 * 1024,
        ),
    )(x, residual, nw, w_fp8)

    bm2 = min(1024, n)
    bn2 = min(2048, d_out)
    out = pl.pallas_call(
        _matmul_kernel,
        grid=(n // bm2, d_out // bn2),
        in_specs=[
            pl.BlockSpec((bm2, d_in), lambda i, j: (i, 0)),
            pl.BlockSpec((bn2, d_in), lambda i, j: (j, 0)),
            pl.BlockSpec(memory_space=pltpu.SMEM),
        ],
        out_specs=pl.BlockSpec((bm2, bn2), lambda i, j: (i, j)),
        out_shape=jax.ShapeDtypeStruct((n, d_out), jnp.float32),
        compiler_params=pltpu.CompilerParams(
            dimension_semantics=("parallel", "arbitrary"),
            vmem_limit_bytes=56 * 1024 * 1024,
        ),
    )(q8, w8, out_scale)

    return (out, add_out)


# speedup vs baseline: 1.7352x; 1.0333x over previous
import jax
import jax.numpy as jnp
from jax import lax
from jax.experimental import pallas as pl
from jax.experimental.pallas import tpu as pltpu

_EPS = 1e-5
_FP8_MAX = 448.0
_F8 = jnp.float8_e4m3fn

_BM = 1024
_BN = 1024
_CH = 128
_NCH = _BM // _CH


def _norm_chunk(x_ref, r_ref, nw_ref):
    a = x_ref[...] + r_ref[...]
    inv = lax.rsqrt(jnp.mean(a * a, axis=-1, keepdims=True) + _EPS)
    q = jnp.clip(a * inv * nw_ref[...], -_FP8_MAX, _FP8_MAX).astype(_F8)
    return a, q


def _prologue_kernel(x_ref, r_ref, nw_ref, w_ref, add_ref, q0_ref, w8_ref):
    w8_ref[...] = w_ref[...].astype(_F8)

    @pl.when(pl.program_id(0) < _NCH)
    def _():
        a, q = _norm_chunk(x_ref, r_ref, nw_ref)
        add_ref[...] = a
        q0_ref[...] = q


def _fused_kernel(q0_ref, x_ref, r_ref, nw_ref, w_ref, s_ref, add_in_ref,
                  o_ref, add_ref, q_scr):
    del add_in_ref
    i = pl.program_id(0)
    j = pl.program_id(1)
    ni = pl.num_programs(0)

    @pl.when(jnp.logical_and(i == 0, j == 0))
    def _():
        q_scr[0] = q0_ref[...]

    @pl.when(jnp.logical_and(i < ni - 1, j < _NCH))
    def _():
        a, q = _norm_chunk(x_ref, r_ref, nw_ref)
        add_ref[...] = a
        q_scr[lax.rem(i + 1, 2), pl.ds(j * _CH, _CH), :] = q

    qv = q_scr[lax.rem(i, 2)]
    acc = lax.dot_general(
        qv,
        w_ref[...],
        dimension_numbers=(((1,), (1,)), ((), ())),
        preferred_element_type=jnp.float32,
    )
    o_ref[...] = acc * s_ref[0]


def kernel(x, residual, norm_weight, w_fp8, input_scale, w_scale):
    n, d_in = x.shape
    d_out = w_fp8.shape[0]

    nw = (norm_weight / input_scale).astype(jnp.float32).reshape(1, d_in)
    out_scale = (input_scale * w_scale).astype(jnp.float32).reshape(1)

    n_blk = n // _BM
    n_ch_total = n // _CH
    bw = 512
    ga = d_out // bw

    add0, q0, w8 = pl.pallas_call(
        _prologue_kernel,
        grid=(ga,),
        in_specs=[
            pl.BlockSpec((_CH, d_in), lambda k: (jnp.minimum(k, _NCH - 1), 0)),
            pl.BlockSpec((_CH, d_in), lambda k: (jnp.minimum(k, _NCH - 1), 0)),
            pl.BlockSpec((1, d_in), lambda k: (0, 0)),
            pl.BlockSpec((bw, d_in), lambda k: (k, 0)),
        ],
        out_specs=[
            pl.BlockSpec((_CH, d_in), lambda k: (jnp.minimum(k, _NCH - 1), 0)),
            pl.BlockSpec((_CH, d_in), lambda k: (jnp.minimum(k, _NCH - 1), 0)),
            pl.BlockSpec((bw, d_in), lambda k: (k, 0)),
        ],
        out_shape=[
            jax.ShapeDtypeStruct((n, d_in), jnp.float32),
            jax.ShapeDtypeStruct((_BM, d_in), _F8),
            jax.ShapeDtypeStruct((d_out, d_in), _F8),
        ],
        compiler_params=pltpu.CompilerParams(
            dimension_semantics=("arbitrary",),
            vmem_limit_bytes=56 * 1024 * 1024,
        ),
    )(x[:_BM], residual[:_BM], nw, w_fp8)

    def _chunk_idx(i, j):
        return (jnp.minimum((i + 1) * _NCH + jnp.minimum(j, _NCH - 1),
                            n_ch_total - 1), 0)

    out, add_out = pl.pallas_call(
        _fused_kernel,
        grid=(n_blk, d_out // _BN),
        in_specs=[
            pl.BlockSpec((_BM, d_in), lambda i, j: (0, 0)),
            pl.BlockSpec((_CH, d_in), _chunk_idx),
            pl.BlockSpec((_CH, d_in), _chunk_idx),
            pl.BlockSpec((1, d_in), lambda i, j: (0, 0)),
            pl.BlockSpec((_BN, d_in), lambda i, j: (j, 0)),
            pl.BlockSpec(memory_space=pltpu.SMEM),
            pl.BlockSpec(memory_space=pl.ANY),
        ],
        out_specs=[
            pl.BlockSpec((_BM, _BN), lambda i, j: (i, j)),
            pl.BlockSpec((_CH, d_in), _chunk_idx),
        ],
        out_shape=[
            jax.ShapeDtypeStruct((n, d_out), jnp.float32),
            jax.ShapeDtypeStruct((n, d_in), jnp.float32),
        ],
        scratch_shapes=[pltpu.VMEM((2, _BM, d_in), _F8)],
        input_output_aliases={6: 1},
        compiler_params=pltpu.CompilerParams(
            dimension_semantics=("arbitrary", "arbitrary"),
            vmem_limit_bytes=56 * 1024 * 1024,
        ),
    )(q0, x, residual, nw, w8, out_scale, add0)

    return (out, add_out)
